# baseline (speedup 1.0000x reference)
import jax
import jax.numpy as jnp
from jax.experimental import pallas as pl


_ROWS = 4096
_SEQ = 50
_NUM = 256
_BLOCK_ROWS = 128


def _onehot_block(idx_ref, out_ref):
    idx = idx_ref[...]
    iota = jax.lax.broadcasted_iota(jnp.int32, (_BLOCK_ROWS, _SEQ, _NUM), 2)
    out_ref[...] = (idx[:, :, None] == iota).astype(jnp.float32)


def kernel(input_tensor):
    grid = (_ROWS // _BLOCK_ROWS,)
    return pl.pallas_call(
        _onehot_block,
        grid=grid,
        in_specs=[pl.BlockSpec((_BLOCK_ROWS, _SEQ), lambda i: (i, 0))],
        out_specs=pl.BlockSpec((_BLOCK_ROWS, _SEQ, _NUM), lambda i: (i, 0, 0)),
        out_shape=jax.ShapeDtypeStruct((_ROWS, _SEQ, _NUM), jnp.float32),
    )(input_tensor)
